# Initial kernel scaffold; baseline (speedup 1.0000x reference)
import jax
import jax.numpy as jnp
from jax.experimental import pallas as pl


def kernel(pred_grasps, confidence, pred_widths, points):
    raise NotImplementedError("write your pallas kernel here")


# speedup vs baseline: 9.5749x; 9.5749x over previous
import functools

import jax
import jax.numpy as jnp
from jax import lax
from jax.experimental import pallas as pl
from jax.experimental.pallas import tpu as pltpu
from jax.experimental.pallas import tpu_sc as plsc

N = 128 * 4096
NSUB = 16
CHUNK = N // NSUB
PHASE1 = 512
BIG = 1 << 26


def _allreduce_min(v):
    m = v[0]
    for i in range(1, 16):
        m = jnp.minimum(m, v[i])
    return m


def _sc_body(conf_hbm, pg_hbm, out_conf, out_pg,
             buf, candv, sbuf, cbuf, pgbuf, ovec, cand_hbm):
    sid = lax.axis_index("s")
    base = pl.multiple_of(sid * CHUNK, CHUNK)

    def scan(lo, n):
        def body(j, acc):
            v = buf[pl.ds(lo + j * 16, 16)]
            gidx = ((base + lo + j * 16).astype(jnp.float32)
                    + lax.iota(jnp.int32, 16).astype(jnp.float32))
            return jnp.minimum(acc, jnp.where(v > 0.0, gidx, jnp.float32(BIG)))
        return lax.fori_loop(0, n // 16, body,
                             jnp.full((16,), BIG, jnp.float32))

    pltpu.sync_copy(conf_hbm.at[pl.ds(base, PHASE1)], buf.at[pl.ds(0, PHASE1)])
    accv1 = scan(0, PHASE1)
    candv[...] = accv1

    @pl.when(_allreduce_min(accv1) >= BIG)
    def _phase2():
        off = pl.multiple_of(base + PHASE1, 8)
        pltpu.sync_copy(conf_hbm.at[pl.ds(off, CHUNK - PHASE1)],
                        buf.at[pl.ds(PHASE1, CHUNK - PHASE1)])
        candv[...] = jnp.minimum(accv1, scan(PHASE1, CHUNK - PHASE1))

    pltpu.sync_copy(candv, cand_hbm.at[sid])
    plsc.subcore_barrier()

    @pl.when(sid == 0)
    def _finalize():
        pltpu.sync_copy(cand_hbm, sbuf)
        m = sbuf[0]
        for i in range(1, NSUB):
            m = jnp.minimum(m, sbuf[i])
        gmin = _allreduce_min(m)
        idx = jnp.where(gmin >= N, jnp.float32(0.0), gmin).astype(jnp.int32)

        al = jnp.minimum((idx >> 3) << 3, jnp.int32(N - 16))
        al = pl.multiple_of(al, 8)
        pltpu.sync_copy(conf_hbm.at[pl.ds(al, 16)], cbuf)
        lane = idx - al
        sigv = 1.0 / (1.0 + jnp.exp(-cbuf[...]))
        val = sigv[0]
        for i in range(1, 16):
            val = jnp.where(lane == i, sigv[i], val)
        ovec[...] = jnp.where(lax.iota(jnp.int32, 16) == 0, val, 0.0)
        pltpu.sync_copy(ovec, out_conf)

        pltpu.sync_copy(pg_hbm.at[pl.ds(idx, 1)], pgbuf)
        pltpu.sync_copy(pgbuf, out_pg)


@jax.jit
def _first_grasp(conf_flat, pg_flat):
    mesh = plsc.VectorSubcoreMesh(core_axis_name="c", subcore_axis_name="s",
                                  num_cores=1)
    run = pl.kernel(
        _sc_body,
        out_type=(
            jax.ShapeDtypeStruct((16,), jnp.float32),
            jax.ShapeDtypeStruct((1, 16), jnp.float32),
        ),
        mesh=mesh,
        scratch_types=(
            pltpu.VMEM((CHUNK,), jnp.float32),
            pltpu.VMEM((16,), jnp.float32),
            pltpu.VMEM((NSUB, 16), jnp.float32),
            pltpu.VMEM((16,), jnp.float32),
            pltpu.VMEM((1, 16), jnp.float32),
            pltpu.VMEM((16,), jnp.float32),
            pltpu.HBM((NSUB, 16), jnp.float32),
        ),
    )
    return run(conf_flat, pg_flat)


def kernel(pred_grasps, confidence, pred_widths, points):
    conf_flat = confidence.reshape(-1)
    pg_flat = pred_grasps.reshape(-1, 16)
    out_conf, out_pg = _first_grasp(conf_flat, pg_flat)
    return out_pg.reshape(1, 4, 4), out_conf[0:1]
